# baseline (speedup 1.0000x reference)
import dataclasses
import functools

import jax
import jax.numpy as jnp
from jax.experimental import pallas as pl
from jax.experimental.pallas import tpu as pltpu
from jax.experimental.pallas import tpu_sc as plsc

N = 10000
E = 320000
D = 128

NC = 2
NS = 16
NW = NC * NS
B = 128
E_PAD = 327680
TOTAL_CHUNKS = E_PAD // B
CH0 = 160
CH1 = TOTAL_CHUNKS // NS - CH0
N_PAD = 10240
ROWS_PER_TILE = N_PAD // NS


def _make_sc_kernel():
    mesh = plsc.VectorSubcoreMesh(core_axis_name="c", subcore_axis_name="s")
    cp = pltpu.CompilerParams()
    if "needs_layout_passes" in pltpu.CompilerParams.__dataclass_fields__:
        cp = dataclasses.replace(cp, needs_layout_passes=False)

    @functools.partial(
        pl.kernel,
        compiler_params=cp,
        out_type=jax.ShapeDtypeStruct((NC, N_PAD, D), jnp.float32),
        mesh=mesh,
        scratch_types=[
            pltpu.VMEM((4, 4, B), jnp.int32),
            pltpu.VMEM((B,), jnp.float32),
            pltpu.VMEM((B, D), jnp.float32),
            pltpu.VMEM((B, D), jnp.float32),
            pltpu.VMEM((16,), jnp.float32),
            pltpu.VMEM_SHARED((N_PAD, D), jnp.float32),
            pltpu.SemaphoreType.DMA,
            pltpu.SemaphoreType.DMA,
            pltpu.SemaphoreType.DMA,
            pltpu.SemaphoreType.DMA,
            pltpu.SemaphoreType.DMA,
            pltpu.SemaphoreType.DMA,
            pltpu.SemaphoreType.DMA,
            pltpu.SemaphoreType.DMA,
        ],
    )
    def sc_kernel(h_hbm, meta_hbm, neglam_hbm, out_hbm,
                  meta_v, w_v, rows0, rows1, neglam_v, acc,
                  sem_m0, sem_m1, sem_m2, sem_m3,
                  sem_g0, sem_g1, sem_s0, sem_s1):
        c = jax.lax.axis_index("c")
        s = jax.lax.axis_index("s")
        rows = (rows0, rows1)
        sem_m = (sem_m0, sem_m1, sem_m2, sem_m3)
        sem_g = (sem_g0, sem_g1)
        sem_s = (sem_s0, sem_s1)

        start = jnp.where(c == 0, s * CH0, NS * CH0 + s * CH1)
        niter = jnp.where(c == 0, CH0 // 4, CH1 // 4)

        count = niter * 4

        @pl.when(count > 0)
        def _():
            for u in range(3):
                pltpu.async_copy(meta_hbm.at[start + u], meta_v.at[u],
                                 sem_m[u])

        zeros16 = jnp.zeros((16,), jnp.float32)

        @pl.loop(0, B)
        def _(i):
            for j in range(D // 16):
                rows0.at[i][pl.ds(j * 16, 16)] = zeros16

        base_row = s * ROWS_PER_TILE
        for i in range(ROWS_PER_TILE // B):
            pltpu.sync_copy(rows0, acc.at[pl.ds(base_row + i * B, B)])

        pltpu.sync_copy(neglam_hbm, neglam_v)
        neglam = neglam_v[...]

        @pl.when(count > 0)
        def _():
            pltpu.make_async_copy(meta_hbm.at[start], meta_v.at[0],
                                  sem_m[0]).wait()
        plsc.subcore_barrier()

        @pl.when(count > 0)
        def _():
            pltpu.async_copy(h_hbm.at[meta_v.at[0, 0]], rows0, sem_g0)

        def do_chunk(q, b, m):
            k = start + q
            rb, ro = rows[b], rows[1 - b]
            mp = (m + 3) % 4
            mn = (m + 1) % 4

            @pl.when(q > 0)
            def _():
                pltpu.make_async_copy(
                    ro, acc.at[meta_v.at[mp, 1]], sem_s[1 - b]).wait()

            @pl.when(q + 3 < count)
            def _():
                pltpu.async_copy(meta_hbm.at[k + 3], meta_v.at[mp],
                                 sem_m[mp])

            @pl.when(q + 1 < count)
            def _():
                pltpu.make_async_copy(meta_hbm.at[k + 1],
                                      meta_v.at[mn], sem_m[mn]).wait()
                pltpu.async_copy(h_hbm.at[meta_v.at[mn, 0]], ro,
                                 sem_g[1 - b])

            for g in range(B // 16):
                dt_bits = meta_v.at[m, 2][pl.ds(g * 16, 16)]
                nm_bits = meta_v.at[m, 3][pl.ds(g * 16, 16)]
                dt_f = plsc.bitcast(dt_bits, jnp.float32)
                nm_f = plsc.bitcast(nm_bits, jnp.float32)
                w_v[pl.ds(g * 16, 16)] = nm_f * jnp.exp(dt_f * neglam)

            pltpu.make_async_copy(h_hbm.at[meta_v.at[m, 0]], rb,
                                  sem_g[b]).wait()

            @pl.loop(0, B // 16)
            def _(g):
                wg = w_v[pl.ds(g * 16, 16)]
                for e in range(16):
                    ws = wg[e]
                    row = g * 16 + e
                    for j in range(D // 16):
                        sl = pl.ds(j * 16, 16)
                        rb.at[row][sl] = rb.at[row][sl] * ws

            pltpu.async_copy(rb, acc.at[meta_v.at[m, 1]], sem_s[b],
                             add=True)

        @pl.loop(0, niter)
        def _(kk):
            for u in range(4):
                do_chunk(4 * kk + u, u % 2, u)

        @pl.when(count > 0)
        def _():
            pltpu.make_async_copy(
                rows1, acc.at[meta_v.at[3, 1]], sem_s1).wait()
        plsc.subcore_barrier()

        pltpu.sync_copy(acc.at[pl.ds(base_row, ROWS_PER_TILE)],
                        out_hbm.at[c].at[pl.ds(base_row, ROWS_PER_TILE)])

    return sc_kernel


_sc_kernel_cache = []


def _get_sc_kernel():
    if not _sc_kernel_cache:
        _sc_kernel_cache.append(_make_sc_kernel())
    return _sc_kernel_cache[0]


def _combine_body(p_ref, o_ref):
    o_ref[...] = p_ref[0, :N] + p_ref[1, :N]


def _combine(parts):
    return pl.pallas_call(
        _combine_body,
        out_shape=jax.ShapeDtypeStruct((N, D), jnp.float32),
    )(parts)


@jax.jit
def kernel(h, edge_index, dt, norm, decay_lam):
    src = edge_index[0]
    dst = edge_index[1]
    pad = E_PAD - E
    srcp = jnp.concatenate([src, jnp.zeros((pad,), jnp.int32)])
    dstp = jnp.concatenate([dst, jnp.zeros((pad,), jnp.int32)])
    dtp = jnp.concatenate([dt, jnp.zeros((pad,), jnp.float32)])
    nmp = jnp.concatenate([norm, jnp.zeros((pad,), jnp.float32)])
    dt_bits = jax.lax.bitcast_convert_type(dtp, jnp.int32)
    nm_bits = jax.lax.bitcast_convert_type(nmp, jnp.int32)
    meta = jnp.stack([srcp, dstp, dt_bits, nm_bits])
    meta = meta.reshape(4, TOTAL_CHUNKS, B).transpose(1, 0, 2)
    neg_lam = -(jax.nn.relu(decay_lam) + jnp.float32(1e-4))
    neglam_arr = jnp.full((16,), neg_lam, jnp.float32)
    parts = _get_sc_kernel()(h, meta, neglam_arr)
    return _combine(parts)


# speedup vs baseline: 1.7586x; 1.7586x over previous
import dataclasses
import functools

import jax
import jax.numpy as jnp
from jax.experimental import pallas as pl
from jax.experimental.pallas import tpu as pltpu
from jax.experimental.pallas import tpu_sc as plsc

N = 10000
E = 320000
D = 128

NC = 2
NS = 16
NW = NC * NS
B = 128
E_PAD = 327680
TOTAL_CHUNKS = E_PAD // B
CH0 = 152
CH1 = TOTAL_CHUNKS // NS - CH0
N_PAD = 10240
ROWS_PER_TILE = N_PAD // NS


def _make_sc_kernel():
    mesh = plsc.VectorSubcoreMesh(core_axis_name="c", subcore_axis_name="s")
    cp = pltpu.CompilerParams()
    if "needs_layout_passes" in pltpu.CompilerParams.__dataclass_fields__:
        cp = dataclasses.replace(cp, needs_layout_passes=False)

    @functools.partial(
        pl.kernel,
        compiler_params=cp,
        out_type=jax.ShapeDtypeStruct((NC, N_PAD, D), jnp.float32),
        mesh=mesh,
        scratch_types=[
            pltpu.VMEM((4, 4, B), jnp.int32),
            pltpu.VMEM((B,), jnp.float32),
            pltpu.VMEM((B, D), jnp.float32),
            pltpu.VMEM((B, D), jnp.float32),
            pltpu.VMEM((16,), jnp.float32),
            pltpu.VMEM_SHARED((N_PAD, D), jnp.float32),
            pltpu.SemaphoreType.DMA,
            pltpu.SemaphoreType.DMA,
            pltpu.SemaphoreType.DMA,
            pltpu.SemaphoreType.DMA,
            pltpu.SemaphoreType.DMA,
            pltpu.SemaphoreType.DMA,
            pltpu.SemaphoreType.DMA,
            pltpu.SemaphoreType.DMA,
        ],
    )
    def sc_kernel(h_hbm, meta_hbm, neglam_hbm, out_hbm,
                  meta_v, w_v, rows0, rows1, neglam_v, acc,
                  sem_m0, sem_m1, sem_m2, sem_m3,
                  sem_g0, sem_g1, sem_s0, sem_s1):
        c = jax.lax.axis_index("c")
        s = jax.lax.axis_index("s")
        rows = (rows0, rows1)
        sem_m = (sem_m0, sem_m1, sem_m2, sem_m3)
        sem_g = (sem_g0, sem_g1)
        sem_s = (sem_s0, sem_s1)

        start = jnp.where(c == 0, s * CH0, NS * CH0 + s * CH1)
        niter = jnp.where(c == 0, CH0 // 4, CH1 // 4)

        count = niter * 4

        @pl.when(count > 0)
        def _():
            for u in range(3):
                pltpu.async_copy(meta_hbm.at[start + u], meta_v.at[u],
                                 sem_m[u])

        zeros16 = jnp.zeros((16,), jnp.float32)

        @pl.loop(0, B)
        def _(i):
            for j in range(D // 16):
                rows0.at[i][pl.ds(j * 16, 16)] = zeros16

        base_row = s * ROWS_PER_TILE
        for i in range(ROWS_PER_TILE // B):
            pltpu.sync_copy(rows0, acc.at[pl.ds(base_row + i * B, B)])

        pltpu.sync_copy(neglam_hbm, neglam_v)
        neglam = neglam_v[...]

        @pl.when(count > 0)
        def _():
            pltpu.make_async_copy(meta_hbm.at[start], meta_v.at[0],
                                  sem_m[0]).wait()
        plsc.subcore_barrier()

        @pl.when(count > 0)
        def _():
            pltpu.async_copy(h_hbm.at[meta_v.at[0, 0]], rows0, sem_g0)

        def do_chunk(q, b, m):
            k = start + q
            rb, ro = rows[b], rows[1 - b]
            mp = (m + 3) % 4
            mn = (m + 1) % 4

            @pl.when(q > 0)
            def _():
                pltpu.make_async_copy(
                    ro, acc.at[meta_v.at[mp, 1]], sem_s[1 - b]).wait()

            @pl.when(q + 3 < count)
            def _():
                pltpu.async_copy(meta_hbm.at[k + 3], meta_v.at[mp],
                                 sem_m[mp])

            @pl.when(q + 1 < count)
            def _():
                pltpu.make_async_copy(meta_hbm.at[k + 1],
                                      meta_v.at[mn], sem_m[mn]).wait()
                pltpu.async_copy(h_hbm.at[meta_v.at[mn, 0]], ro,
                                 sem_g[1 - b])

            for g in range(B // 16):
                dt_bits = meta_v.at[m, 2][pl.ds(g * 16, 16)]
                nm_bits = meta_v.at[m, 3][pl.ds(g * 16, 16)]
                dt_f = plsc.bitcast(dt_bits, jnp.float32)
                nm_f = plsc.bitcast(nm_bits, jnp.float32)
                w_v[pl.ds(g * 16, 16)] = nm_f * jnp.exp(dt_f * neglam)

            pltpu.make_async_copy(h_hbm.at[meta_v.at[m, 0]], rb,
                                  sem_g[b]).wait()

            @pl.loop(0, B // 16)
            def _(g):
                wg = w_v[pl.ds(g * 16, 16)]
                for e in range(16):
                    ws = wg[e]
                    row = g * 16 + e
                    for j in range(D // 16):
                        sl = pl.ds(j * 16, 16)
                        rb.at[row][sl] = rb.at[row][sl] * ws

            pltpu.async_copy(rb, acc.at[meta_v.at[m, 1]], sem_s[b],
                             add=True)

        @pl.loop(0, niter)
        def _(kk):
            for u in range(4):
                do_chunk(4 * kk + u, u % 2, u)

        @pl.when(count > 0)
        def _():
            pltpu.make_async_copy(
                rows1, acc.at[meta_v.at[3, 1]], sem_s1).wait()
        plsc.subcore_barrier()

        pltpu.sync_copy(acc.at[pl.ds(base_row, ROWS_PER_TILE)],
                        out_hbm.at[c].at[pl.ds(base_row, ROWS_PER_TILE)])

    return sc_kernel


_sc_kernel_cache = []


def _get_sc_kernel():
    if not _sc_kernel_cache:
        _sc_kernel_cache.append(_make_sc_kernel())
    return _sc_kernel_cache[0]


def _combine_body(p_ref, o_ref):
    o_ref[...] = p_ref[0, :N] + p_ref[1, :N]


def _combine(parts):
    return pl.pallas_call(
        _combine_body,
        out_shape=jax.ShapeDtypeStruct((N, D), jnp.float32),
    )(parts)


@jax.jit
def kernel(h, edge_index, dt, norm, decay_lam):
    src = edge_index[0]
    dst = edge_index[1]
    pad = E_PAD - E
    srcp = jnp.concatenate([src, jnp.zeros((pad,), jnp.int32)])
    dstp = jnp.concatenate([dst, jnp.zeros((pad,), jnp.int32)])
    dtp = jnp.concatenate([dt, jnp.zeros((pad,), jnp.float32)])
    nmp = jnp.concatenate([norm, jnp.zeros((pad,), jnp.float32)])
    dt_bits = jax.lax.bitcast_convert_type(dtp, jnp.int32)
    nm_bits = jax.lax.bitcast_convert_type(nmp, jnp.int32)
    meta = jnp.stack([srcp, dstp, dt_bits, nm_bits])
    meta = meta.reshape(4, TOTAL_CHUNKS, B).transpose(1, 0, 2)
    neg_lam = -(jax.nn.relu(decay_lam) + jnp.float32(1e-4))
    neglam_arr = jnp.full((16,), neg_lam, jnp.float32)
    parts = _get_sc_kernel()(h, meta, neglam_arr)
    return _combine(parts)
